# Initial kernel scaffold; baseline (speedup 1.0000x reference)
import jax
import jax.numpy as jnp
from jax.experimental import pallas as pl


def kernel(x, temperature, W_qkv, b_qkv, W_dw, b_dw, W_proj, b_proj, attn1, attn2, attn3, attn4):
    raise NotImplementedError("write your pallas kernel here")


# speedup vs baseline: 2.8389x; 2.8389x over previous
import jax
import jax.numpy as jnp
from jax import lax
from jax.experimental import pallas as pl
from jax.experimental.pallas import tpu as pltpu

_NEG = -3.0e38


def _dwconv_into(o_ref, u, wdw_ref, bdw_ref, tt, ww):
    ppl = u.shape[1]
    qi = lax.broadcasted_iota(jnp.int32, (1, tt), 1)
    w = jnp.bitwise_and(qi, ww - 1)
    m0 = jnp.where(w != 0, 1.0, 0.0).astype(jnp.float32)
    m2 = jnp.where(w != ww - 1, 1.0, 0.0).astype(jnp.float32)
    first = True
    for dy in range(3):
        for dx in range(3):
            p0 = dy * ww + dx - 1
            if p0 < 0:
                tap = jnp.concatenate([u[:, :1], u[:, : tt - 1]], axis=1)
            elif p0 + tt > ppl:
                tap = jnp.concatenate([u[:, p0:], u[:, -1:]], axis=1)
            else:
                tap = u[:, p0 : p0 + tt]
            coef = wdw_ref[:, dy * 3 + dx : dy * 3 + dx + 1]
            term = coef * tap
            if dx == 0:
                term = term * m0
            elif dx == 2:
                term = term * m2
            if first:
                o_ref[...] = term + bdw_ref[...]
                first = False
            else:
                o_ref[...] += term


def _row_valid(i, t, hh, ww, pp):
    p = lax.broadcasted_iota(jnp.int32, (1, pp), 1)
    prow = i * t + lax.shift_right_logical(p, ww.bit_length() - 1)
    return jnp.where((prow >= 1) & (prow <= hh), 1.0, 0.0)


def _make_pass1(hh, ww, cin, t):
    nt = hh // t
    tt = t * ww
    pp = (t + 2) * ww

    def body(xf_hbm, wqk_ref, bqk_ref, wdw_ref, bdw_ref, tempc_ref, attn_ref,
             xbuf, qk_ref, qsq, ksq, sem):
        i = pl.program_id(0)
        cp = pltpu.make_async_copy(
            xf_hbm.at[:, pl.ds(i * t * ww, pp)], xbuf, sem)
        cp.start()
        cp.wait()

        @pl.when(i == 0)
        def _():
            attn_ref[...] = jnp.zeros_like(attn_ref)
            qsq[...] = jnp.zeros_like(qsq)
            ksq[...] = jnp.zeros_like(ksq)

        x = xbuf[...]
        u = jnp.dot(wqk_ref[...], x, preferred_element_type=jnp.float32)
        u = u + bqk_ref[...] * _row_valid(i, t, hh, ww, pp)
        _dwconv_into(qk_ref, u, wdw_ref, bdw_ref, tt, ww)
        qf = qk_ref[:cin, :]
        kf = qk_ref[cin:, :]
        dn = (((1,), (1,)), ((), ()))
        attn_ref[...] += lax.dot_general(
            qf, kf, dn, preferred_element_type=jnp.float32)
        qsq[...] += jnp.sum(qf * qf, axis=1, keepdims=True)
        ones1 = jnp.ones((1, tt), jnp.float32)
        ksq[...] += lax.dot_general(
            ones1, kf * kf, dn, preferred_element_type=jnp.float32)

        @pl.when(i == nt - 1)
        def _():
            invq = 1.0 / jnp.maximum(jnp.sqrt(qsq[...]), 1e-12)
            invk = 1.0 / jnp.maximum(jnp.sqrt(ksq[...]), 1e-12)
            attn_ref[...] = attn_ref[...] * invq * invk * tempc_ref[...]

    return pl.pallas_call(
        body,
        grid=(nt,),
        in_specs=[
            pl.BlockSpec(memory_space=pl.ANY),
            pl.BlockSpec((2 * cin, cin), lambda i: (0, 0)),
            pl.BlockSpec((2 * cin, 1), lambda i: (0, 0)),
            pl.BlockSpec((2 * cin, 9), lambda i: (0, 0)),
            pl.BlockSpec((2 * cin, 1), lambda i: (0, 0)),
            pl.BlockSpec((cin, 1), lambda i: (0, 0)),
        ],
        out_specs=pl.BlockSpec((cin, cin), lambda i: (0, 0)),
        out_shape=jax.ShapeDtypeStruct((cin, cin), jnp.float32),
        scratch_shapes=[
            pltpu.VMEM((cin, pp), jnp.float32),
            pltpu.VMEM((2 * cin, tt), jnp.float32),
            pltpu.VMEM((cin, 1), jnp.float32),
            pltpu.VMEM((1, cin), jnp.float32),
            pltpu.SemaphoreType.DMA,
        ],
        compiler_params=pltpu.CompilerParams(
            dimension_semantics=("arbitrary",)),
    )


def _make_pass2(hh, ww, cin, t, cph):
    nt = hh // t
    tt = t * ww
    pp = (t + 2) * ww

    def body(xf_hbm, wv_ref, bv_ref, wdw_ref, bdw_ref, attn_in, wproj_ref,
             bproj_ref, hr_ref, hc_ref, wbr_ref, o_ref, xbuf, v_ref, m_ref,
             sem):
        i = pl.program_id(0)
        cp = pltpu.make_async_copy(
            xf_hbm.at[:, pl.ds(i * t * ww, pp)], xbuf, sem)
        cp.start()
        cp.wait()

        @pl.when(i == 0)
        def _():
            bd = hr_ref[...] == hc_ref[...]
            a = jnp.where(bd, attn_in[...], _NEG)
            x1 = a[:, :, None]
            x2 = a[:, None, :]
            n = a.shape[0]
            lt = (lax.broadcasted_iota(jnp.int32, (1, n, n), 1)
                  < lax.broadcasted_iota(jnp.int32, (1, n, n), 2))
            cnt = (x1 > x2) | ((x1 == x2) & lt)
            rank = jnp.sum(cnt.astype(jnp.float32), axis=1)

            def smax(kk):
                sel = bd & (rank < kk)
                m = jnp.max(jnp.where(sel, a, _NEG), axis=1, keepdims=True)
                e = jnp.where(sel, jnp.exp(a - m), 0.0)
                return e / jnp.sum(e, axis=1, keepdims=True)

            bdm = (wbr_ref[0] * smax(float(cph // 2))
                   + wbr_ref[1] * smax(float(cph * 2 // 3))
                   + (wbr_ref[2] + wbr_ref[3]) * smax(float(cph * 3 // 4)))
            m_ref[...] = jnp.dot(wproj_ref[...], bdm,
                                 preferred_element_type=jnp.float32)

        x = xbuf[...]
        u = jnp.dot(wv_ref[...], x, preferred_element_type=jnp.float32)
        u = u + bv_ref[...] * _row_valid(i, t, hh, ww, pp)
        _dwconv_into(v_ref, u, wdw_ref, bdw_ref, tt, ww)
        o_ref[...] = jnp.dot(m_ref[...], v_ref[...],
                             preferred_element_type=jnp.float32) + bproj_ref[...]

    return pl.pallas_call(
        body,
        grid=(nt,),
        in_specs=[
            pl.BlockSpec(memory_space=pl.ANY),
            pl.BlockSpec((cin, cin), lambda i: (0, 0)),
            pl.BlockSpec((cin, 1), lambda i: (0, 0)),
            pl.BlockSpec((cin, 9), lambda i: (0, 0)),
            pl.BlockSpec((cin, 1), lambda i: (0, 0)),
            pl.BlockSpec((cin, cin), lambda i: (0, 0)),
            pl.BlockSpec((cin, cin), lambda i: (0, 0)),
            pl.BlockSpec((cin, 1), lambda i: (0, 0)),
            pl.BlockSpec((cin, 1), lambda i: (0, 0)),
            pl.BlockSpec((1, cin), lambda i: (0, 0)),
            pl.BlockSpec(memory_space=pltpu.SMEM),
        ],
        out_specs=pl.BlockSpec((cin, tt), lambda i: (0, i)),
        out_shape=jax.ShapeDtypeStruct((cin, hh * ww), jnp.float32),
        scratch_shapes=[
            pltpu.VMEM((cin, pp), jnp.float32),
            pltpu.VMEM((cin, tt), jnp.float32),
            pltpu.VMEM((cin, cin), jnp.float32),
            pltpu.SemaphoreType.DMA,
        ],
        compiler_params=pltpu.CompilerParams(
            dimension_semantics=("arbitrary",)),
    )


def kernel(x, temperature, W_qkv, b_qkv, W_dw, b_dw, W_proj, b_proj,
           attn1, attn2, attn3, attn4):
    _, cin, hh, ww = x.shape
    heads = temperature.shape[0]
    cph = cin // heads
    t = 16 if hh % 16 == 0 else hh
    f32 = jnp.float32

    xp = jnp.pad(x[0], ((0, 0), (1, 1), (0, 0))).reshape(cin, (hh + 2) * ww)
    wqk = W_qkv[: 2 * cin]
    wv = W_qkv[2 * cin :]
    bqk = b_qkv[: 2 * cin].reshape(-1, 1)
    bv = b_qkv[2 * cin :].reshape(-1, 1)
    wdw = W_dw[:, 0].reshape(3 * cin, 9)
    bdwv = b_dw[2 * cin :].reshape(-1, 1)
    tempc = jnp.repeat(temperature.reshape(heads), cph).reshape(cin, 1)
    hr = (jnp.arange(cin, dtype=jnp.int32) // cph).astype(f32).reshape(cin, 1)
    hc = hr.reshape(1, cin)
    wbr = jnp.concatenate([attn1, attn2, attn3, attn4]).astype(f32)

    attn = _make_pass1(hh, ww, cin, t)(
        xp, wqk, bqk, wdw[: 2 * cin], b_dw[: 2 * cin].reshape(-1, 1), tempc)
    out = _make_pass2(hh, ww, cin, t, cph)(
        xp, wv, bv, wdw[2 * cin :], bdwv, attn, W_proj,
        b_proj.reshape(cin, 1), hr, hc, wbr)
    return out.reshape(1, cin, hh, ww)


# speedup vs baseline: 3.1964x; 1.1259x over previous
import jax
import jax.numpy as jnp
from jax import lax
from jax.experimental import pallas as pl
from jax.experimental.pallas import tpu as pltpu

_NEG = -3.0e38


_CW = 2048


def _dwconv_into(o_ref, u, wdw_ref, bdw_ref, tt, ww):
    ppl = u.shape[1]
    cw = min(_CW, tt)
    qi = lax.broadcasted_iota(jnp.int32, (1, cw), 1)
    w = jnp.bitwise_and(qi, ww - 1)
    m0 = jnp.where(w != 0, 1.0, 0.0).astype(jnp.float32)
    m2 = jnp.where(w != ww - 1, 1.0, 0.0).astype(jnp.float32)
    coefs = [wdw_ref[:, j : j + 1] for j in range(9)]
    for c0 in range(0, tt, cw):
        acc = bdw_ref[...] + jnp.zeros((1, cw), jnp.float32)
        for dy in range(3):
            for dx in range(3):
                s0 = c0 + dy * ww + dx - 1
                if s0 < 0:
                    tap = jnp.concatenate([u[:, :1], u[:, : cw - 1]], axis=1)
                elif s0 + cw > ppl:
                    tap = jnp.concatenate([u[:, s0:], u[:, -1:]], axis=1)
                else:
                    tap = u[:, s0 : s0 + cw]
                term = coefs[dy * 3 + dx] * tap
                if dx == 0:
                    term = term * m0
                elif dx == 2:
                    term = term * m2
                acc = acc + term
        o_ref[:, c0 : c0 + cw] = acc


def _row_valid(i, t, hh, ww, pp):
    p = lax.broadcasted_iota(jnp.int32, (1, pp), 1)
    prow = i * t + lax.shift_right_logical(p, ww.bit_length() - 1)
    return jnp.where((prow >= 1) & (prow <= hh), 1.0, 0.0)


def _make_pass1(hh, ww, cin, t):
    nt = hh // t
    tt = t * ww
    pp = (t + 2) * ww

    def body(xf_hbm, wqk_ref, bqk_ref, wdw_ref, bdw_ref, tempc_ref, attn_ref,
             xbuf, qk_ref, qsq, ksq, sem):
        i = pl.program_id(0)

        def start(j):
            s = lax.rem(j, 2)
            pltpu.make_async_copy(
                xf_hbm.at[:, pl.ds(j * t * ww, pp)], xbuf.at[s],
                sem.at[s]).start()

        @pl.when(i == 0)
        def _():
            start(0)
            attn_ref[...] = jnp.zeros_like(attn_ref)
            qsq[...] = jnp.zeros_like(qsq)
            ksq[...] = jnp.zeros_like(ksq)

        @pl.when(i + 1 < nt)
        def _():
            start(i + 1)

        slot = lax.rem(i, 2)
        pltpu.make_async_copy(
            xf_hbm.at[:, pl.ds(i * t * ww, pp)], xbuf.at[slot],
            sem.at[slot]).wait()
        x = xbuf[slot]
        u = jnp.dot(wqk_ref[...], x, preferred_element_type=jnp.float32)
        u = u + bqk_ref[...] * _row_valid(i, t, hh, ww, pp)
        _dwconv_into(qk_ref, u, wdw_ref, bdw_ref, tt, ww)
        qf = qk_ref[:cin, :]
        kf = qk_ref[cin:, :]
        dn = (((1,), (1,)), ((), ()))
        attn_ref[...] += lax.dot_general(
            qf, kf, dn, preferred_element_type=jnp.float32)
        qsq[...] += jnp.sum(qf * qf, axis=1, keepdims=True)
        ones1 = jnp.ones((1, tt), jnp.float32)
        ksq[...] += lax.dot_general(
            ones1, kf * kf, dn, preferred_element_type=jnp.float32)

        @pl.when(i == nt - 1)
        def _():
            invq = 1.0 / jnp.maximum(jnp.sqrt(qsq[...]), 1e-12)
            invk = 1.0 / jnp.maximum(jnp.sqrt(ksq[...]), 1e-12)
            attn_ref[...] = attn_ref[...] * invq * invk * tempc_ref[...]

    return pl.pallas_call(
        body,
        grid=(nt,),
        in_specs=[
            pl.BlockSpec(memory_space=pl.ANY),
            pl.BlockSpec((2 * cin, cin), lambda i: (0, 0)),
            pl.BlockSpec((2 * cin, 1), lambda i: (0, 0)),
            pl.BlockSpec((2 * cin, 9), lambda i: (0, 0)),
            pl.BlockSpec((2 * cin, 1), lambda i: (0, 0)),
            pl.BlockSpec((cin, 1), lambda i: (0, 0)),
        ],
        out_specs=pl.BlockSpec((cin, cin), lambda i: (0, 0)),
        out_shape=jax.ShapeDtypeStruct((cin, cin), jnp.float32),
        scratch_shapes=[
            pltpu.VMEM((2, cin, pp), jnp.float32),
            pltpu.VMEM((2 * cin, tt), jnp.float32),
            pltpu.VMEM((cin, 1), jnp.float32),
            pltpu.VMEM((1, cin), jnp.float32),
            pltpu.SemaphoreType.DMA((2,)),
        ],
        compiler_params=pltpu.CompilerParams(
            dimension_semantics=("arbitrary",)),
    )


def _make_pass2(hh, ww, cin, t, cph):
    nt = hh // t
    tt = t * ww
    pp = (t + 2) * ww

    def body(xf_hbm, wv_ref, bv_ref, wdw_ref, bdw_ref, attn_in, wproj_ref,
             bproj_ref, hr_ref, hc_ref, wbr_ref, o_ref, xbuf, v_ref, m_ref,
             sem):
        i = pl.program_id(0)

        def start(j):
            s = lax.rem(j, 2)
            pltpu.make_async_copy(
                xf_hbm.at[:, pl.ds(j * t * ww, pp)], xbuf.at[s],
                sem.at[s]).start()

        @pl.when(i == 0)
        def _():
            start(0)

        @pl.when(i + 1 < nt)
        def _():
            start(i + 1)

        slot = lax.rem(i, 2)
        pltpu.make_async_copy(
            xf_hbm.at[:, pl.ds(i * t * ww, pp)], xbuf.at[slot],
            sem.at[slot]).wait()

        @pl.when(i == 0)
        def _():
            bd = hr_ref[...] == hc_ref[...]
            a = jnp.where(bd, attn_in[...], _NEG)
            x1 = a[:, :, None]
            x2 = a[:, None, :]
            n = a.shape[0]
            lt = (lax.broadcasted_iota(jnp.int32, (1, n, n), 1)
                  < lax.broadcasted_iota(jnp.int32, (1, n, n), 2))
            cnt = (x1 > x2) | ((x1 == x2) & lt)
            rank = jnp.sum(cnt.astype(jnp.float32), axis=1)

            def smax(kk):
                sel = bd & (rank < kk)
                m = jnp.max(jnp.where(sel, a, _NEG), axis=1, keepdims=True)
                e = jnp.where(sel, jnp.exp(a - m), 0.0)
                return e / jnp.sum(e, axis=1, keepdims=True)

            bdm = (wbr_ref[0] * smax(float(cph // 2))
                   + wbr_ref[1] * smax(float(cph * 2 // 3))
                   + (wbr_ref[2] + wbr_ref[3]) * smax(float(cph * 3 // 4)))
            m_ref[...] = jnp.dot(wproj_ref[...], bdm,
                                 preferred_element_type=jnp.float32)

        x = xbuf[slot]
        u = jnp.dot(wv_ref[...], x, preferred_element_type=jnp.float32)
        u = u + bv_ref[...] * _row_valid(i, t, hh, ww, pp)
        _dwconv_into(v_ref, u, wdw_ref, bdw_ref, tt, ww)
        o_ref[...] = jnp.dot(m_ref[...], v_ref[...],
                             preferred_element_type=jnp.float32) + bproj_ref[...]

    return pl.pallas_call(
        body,
        grid=(nt,),
        in_specs=[
            pl.BlockSpec(memory_space=pl.ANY),
            pl.BlockSpec((cin, cin), lambda i: (0, 0)),
            pl.BlockSpec((cin, 1), lambda i: (0, 0)),
            pl.BlockSpec((cin, 9), lambda i: (0, 0)),
            pl.BlockSpec((cin, 1), lambda i: (0, 0)),
            pl.BlockSpec((cin, cin), lambda i: (0, 0)),
            pl.BlockSpec((cin, cin), lambda i: (0, 0)),
            pl.BlockSpec((cin, 1), lambda i: (0, 0)),
            pl.BlockSpec((cin, 1), lambda i: (0, 0)),
            pl.BlockSpec((1, cin), lambda i: (0, 0)),
            pl.BlockSpec(memory_space=pltpu.SMEM),
        ],
        out_specs=pl.BlockSpec((cin, tt), lambda i: (0, i)),
        out_shape=jax.ShapeDtypeStruct((cin, hh * ww), jnp.float32),
        scratch_shapes=[
            pltpu.VMEM((2, cin, pp), jnp.float32),
            pltpu.VMEM((cin, tt), jnp.float32),
            pltpu.VMEM((cin, cin), jnp.float32),
            pltpu.SemaphoreType.DMA((2,)),
        ],
        compiler_params=pltpu.CompilerParams(
            dimension_semantics=("arbitrary",)),
    )


def kernel(x, temperature, W_qkv, b_qkv, W_dw, b_dw, W_proj, b_proj,
           attn1, attn2, attn3, attn4):
    _, cin, hh, ww = x.shape
    heads = temperature.shape[0]
    cph = cin // heads
    t = 32 if hh % 32 == 0 else hh
    f32 = jnp.float32

    xp = jnp.pad(x[0], ((0, 0), (1, 1), (0, 0))).reshape(cin, (hh + 2) * ww)
    wqk = W_qkv[: 2 * cin]
    wv = W_qkv[2 * cin :]
    bqk = b_qkv[: 2 * cin].reshape(-1, 1)
    bv = b_qkv[2 * cin :].reshape(-1, 1)
    wdw = W_dw[:, 0].reshape(3 * cin, 9)
    bdwv = b_dw[2 * cin :].reshape(-1, 1)
    tempc = jnp.repeat(temperature.reshape(heads), cph).reshape(cin, 1)
    hr = (jnp.arange(cin, dtype=jnp.int32) // cph).astype(f32).reshape(cin, 1)
    hc = hr.reshape(1, cin)
    wbr = jnp.concatenate([attn1, attn2, attn3, attn4]).astype(f32)

    attn = _make_pass1(hh, ww, cin, t)(
        xp, wqk, bqk, wdw[: 2 * cin], b_dw[: 2 * cin].reshape(-1, 1), tempc)
    out = _make_pass2(hh, ww, cin, t, cph)(
        xp, wv, bv, wdw[2 * cin :], bdwv, attn, W_proj,
        b_proj.reshape(cin, 1), hr, hc, wbr)
    return out.reshape(1, cin, hh, ww)


# speedup vs baseline: 3.6120x; 1.1300x over previous
import jax
import jax.numpy as jnp
from jax import lax
from jax.experimental import pallas as pl
from jax.experimental.pallas import tpu as pltpu

_NEG = -3.0e38
_CW = 2048


def _dwconv_into(o_ref, u, wdw_ref, bdw_ref, tt, ww, hb):
    ppl = u.shape[1]
    cw = min(_CW, tt)
    qi = lax.broadcasted_iota(jnp.int32, (1, cw), 1)
    w = jnp.bitwise_and(qi, ww - 1)
    m0 = jnp.where(w != 0, 1.0, 0.0).astype(jnp.float32)
    m2 = jnp.where(w != ww - 1, 1.0, 0.0).astype(jnp.float32)
    coefs = [wdw_ref[:, j : j + 1] for j in range(9)]
    for c0 in range(0, tt, cw):
        acc = bdw_ref[...] + jnp.zeros((1, cw), jnp.float32)
        for dy in range(3):
            for dx in range(3):
                s0 = c0 + (hb - 1 + dy) * ww + dx - 1
                if s0 < 0:
                    tap = jnp.concatenate([u[:, :1], u[:, : cw - 1]], axis=1)
                elif s0 + cw > ppl:
                    tap = jnp.concatenate([u[:, s0:], u[:, -1:]], axis=1)
                else:
                    tap = u[:, s0 : s0 + cw]
                term = coefs[dy * 3 + dx] * tap
                if dx == 0:
                    term = term * m0
                elif dx == 2:
                    term = term * m2
                acc = acc + term
        o_ref[:, c0 : c0 + cw] = acc


def _row_valid(i, t, hh, ww, pp, hb):
    p = lax.broadcasted_iota(jnp.int32, (1, pp), 1)
    prow = i * t + lax.shift_right_logical(p, ww.bit_length() - 1)
    return jnp.where((prow >= hb) & (prow < hh + hb), 1.0, 0.0)


_HB = 8


def _halo_copies(x_hbm, xbuf, sem, j, t, nt):
    s = lax.rem(j, 2)
    hb = _HB
    cps = [(None,
            pltpu.make_async_copy(x_hbm.at[:, pl.ds(j * t, t), :],
                                  xbuf.at[s, :, pl.ds(hb, t), :], sem.at[s]))]
    if nt > 1:
        cps.append((j > 0,
                    pltpu.make_async_copy(x_hbm.at[:, pl.ds(j * t - hb, hb), :],
                                          xbuf.at[s, :, pl.ds(0, hb), :],
                                          sem.at[s])))
        cps.append((j < nt - 1,
                    pltpu.make_async_copy(x_hbm.at[:, pl.ds(j * t + t, hb), :],
                                          xbuf.at[s, :, pl.ds(t + hb, hb), :],
                                          sem.at[s])))
    return cps


def _start_window(x_hbm, xbuf, sem, j, t, nt):
    for cond, cp in _halo_copies(x_hbm, xbuf, sem, j, t, nt):
        if cond is None:
            cp.start()
        else:
            pl.when(cond)(cp.start)


def _wait_window(x_hbm, xbuf, sem, j, t, nt):
    for cond, cp in _halo_copies(x_hbm, xbuf, sem, j, t, nt):
        if cond is None:
            cp.wait()
        else:
            pl.when(cond)(cp.wait)


def _init_halo_rows(xbuf, t, nt, cin, ww):
    hb = _HB
    z = jnp.zeros((cin, hb, ww), jnp.float32)
    xbuf[0, :, 0:hb, :] = z
    if nt <= 2:
        xbuf[(nt - 1) % 2, :, t + hb : t + 2 * hb, :] = z


def _make_pass1(hh, ww, cin, t):
    nt = hh // t
    tt = t * ww
    pp = (t + 2 * _HB) * ww

    def body(x_hbm, wqk_ref, bqk_ref, wdw_ref, bdw_ref, tempc_ref, attn_ref,
             xbuf, qk_ref, qsq, ksq, sem):
        i = pl.program_id(0)

        @pl.when(i == 0)
        def _():
            _init_halo_rows(xbuf, t, nt, cin, ww)
            _start_window(x_hbm, xbuf, sem, 0, t, nt)
            attn_ref[...] = jnp.zeros_like(attn_ref)
            qsq[...] = jnp.zeros_like(qsq)
            ksq[...] = jnp.zeros_like(ksq)

        @pl.when(i + 1 < nt)
        def _():
            _start_window(x_hbm, xbuf, sem, i + 1, t, nt)

        _wait_window(x_hbm, xbuf, sem, i, t, nt)
        x = xbuf[lax.rem(i, 2)].reshape(cin, pp)
        u = jnp.dot(wqk_ref[...], x, preferred_element_type=jnp.float32)
        u = (u + bqk_ref[...]) * _row_valid(i, t, hh, ww, pp, _HB)
        _dwconv_into(qk_ref, u, wdw_ref, bdw_ref, tt, ww, _HB)
        qf = qk_ref[:cin, :]
        kf = qk_ref[cin:, :]
        dn = (((1,), (1,)), ((), ()))
        attn_ref[...] += lax.dot_general(
            qf, kf, dn, preferred_element_type=jnp.float32)
        qsq[...] += jnp.sum(qf * qf, axis=1, keepdims=True)
        ones1 = jnp.ones((1, tt), jnp.float32)
        ksq[...] += lax.dot_general(
            ones1, kf * kf, dn, preferred_element_type=jnp.float32)

        @pl.when(i == nt - 1)
        def _():
            invq = 1.0 / jnp.maximum(jnp.sqrt(qsq[...]), 1e-12)
            invk = 1.0 / jnp.maximum(jnp.sqrt(ksq[...]), 1e-12)
            attn_ref[...] = attn_ref[...] * invq * invk * tempc_ref[...]

    return pl.pallas_call(
        body,
        grid=(nt,),
        in_specs=[
            pl.BlockSpec(memory_space=pl.ANY),
            pl.BlockSpec((2 * cin, cin), lambda i: (0, 0)),
            pl.BlockSpec((2 * cin, 1), lambda i: (0, 0)),
            pl.BlockSpec((2 * cin, 9), lambda i: (0, 0)),
            pl.BlockSpec((2 * cin, 1), lambda i: (0, 0)),
            pl.BlockSpec((cin, 1), lambda i: (0, 0)),
        ],
        out_specs=pl.BlockSpec((cin, cin), lambda i: (0, 0)),
        out_shape=jax.ShapeDtypeStruct((cin, cin), jnp.float32),
        scratch_shapes=[
            pltpu.VMEM((2, cin, t + 2 * _HB, ww), jnp.float32),
            pltpu.VMEM((2 * cin, tt), jnp.float32),
            pltpu.VMEM((cin, 1), jnp.float32),
            pltpu.VMEM((1, cin), jnp.float32),
            pltpu.SemaphoreType.DMA((2,)),
        ],
        compiler_params=pltpu.CompilerParams(
            dimension_semantics=("arbitrary",)),
    )


def _make_pass2(hh, ww, cin, t, cph):
    nt = hh // t
    tt = t * ww
    pp = (t + 2 * _HB) * ww

    def body(x_hbm, wv_ref, bv_ref, wdw_ref, bdw_ref, attn_in, wproj_ref,
             bproj_ref, hr_ref, hc_ref, wbr_ref, o_ref, xbuf, v_ref, m_ref,
             sem):
        i = pl.program_id(0)

        @pl.when(i == 0)
        def _():
            _init_halo_rows(xbuf, t, nt, cin, ww)
            _start_window(x_hbm, xbuf, sem, 0, t, nt)

        @pl.when(i + 1 < nt)
        def _():
            _start_window(x_hbm, xbuf, sem, i + 1, t, nt)

        _wait_window(x_hbm, xbuf, sem, i, t, nt)

        @pl.when(i == 0)
        def _():
            bd = hr_ref[...] == hc_ref[...]
            a = jnp.where(bd, attn_in[...], _NEG)
            x1 = a[:, :, None]
            x2 = a[:, None, :]
            n = a.shape[0]
            lt = (lax.broadcasted_iota(jnp.int32, (1, n, n), 1)
                  < lax.broadcasted_iota(jnp.int32, (1, n, n), 2))
            cnt = (x1 > x2) | ((x1 == x2) & lt)
            rank = jnp.sum(cnt.astype(jnp.float32), axis=1)

            def smax(kk):
                sel = bd & (rank < kk)
                m = jnp.max(jnp.where(sel, a, _NEG), axis=1, keepdims=True)
                e = jnp.where(sel, jnp.exp(a - m), 0.0)
                return e / jnp.sum(e, axis=1, keepdims=True)

            bdm = (wbr_ref[0] * smax(float(cph // 2))
                   + wbr_ref[1] * smax(float(cph * 2 // 3))
                   + (wbr_ref[2] + wbr_ref[3]) * smax(float(cph * 3 // 4)))
            m_ref[...] = jnp.dot(wproj_ref[...], bdm,
                                 preferred_element_type=jnp.float32)

        x = xbuf[lax.rem(i, 2)].reshape(cin, pp)
        u = jnp.dot(wv_ref[...], x, preferred_element_type=jnp.float32)
        u = (u + bv_ref[...]) * _row_valid(i, t, hh, ww, pp, _HB)
        _dwconv_into(v_ref, u, wdw_ref, bdw_ref, tt, ww, _HB)
        o = jnp.dot(m_ref[...], v_ref[...],
                    preferred_element_type=jnp.float32) + bproj_ref[...]
        o_ref[...] = o.reshape(cin, t, ww)

    return pl.pallas_call(
        body,
        grid=(nt,),
        in_specs=[
            pl.BlockSpec(memory_space=pl.ANY),
            pl.BlockSpec((cin, cin), lambda i: (0, 0)),
            pl.BlockSpec((cin, 1), lambda i: (0, 0)),
            pl.BlockSpec((cin, 9), lambda i: (0, 0)),
            pl.BlockSpec((cin, 1), lambda i: (0, 0)),
            pl.BlockSpec((cin, cin), lambda i: (0, 0)),
            pl.BlockSpec((cin, cin), lambda i: (0, 0)),
            pl.BlockSpec((cin, 1), lambda i: (0, 0)),
            pl.BlockSpec((cin, 1), lambda i: (0, 0)),
            pl.BlockSpec((1, cin), lambda i: (0, 0)),
            pl.BlockSpec(memory_space=pltpu.SMEM),
        ],
        out_specs=pl.BlockSpec((cin, t, ww), lambda i: (0, i, 0)),
        out_shape=jax.ShapeDtypeStruct((cin, hh, ww), jnp.float32),
        scratch_shapes=[
            pltpu.VMEM((2, cin, t + 2 * _HB, ww), jnp.float32),
            pltpu.VMEM((cin, tt), jnp.float32),
            pltpu.VMEM((cin, cin), jnp.float32),
            pltpu.SemaphoreType.DMA((2,)),
        ],
        compiler_params=pltpu.CompilerParams(
            dimension_semantics=("arbitrary",)),
    )


def kernel(x, temperature, W_qkv, b_qkv, W_dw, b_dw, W_proj, b_proj,
           attn1, attn2, attn3, attn4):
    _, cin, hh, ww = x.shape
    heads = temperature.shape[0]
    cph = cin // heads
    t = 32 if hh % 32 == 0 else hh
    f32 = jnp.float32

    x0 = x[0]
    wqk = W_qkv[: 2 * cin]
    wv = W_qkv[2 * cin :]
    bqk = b_qkv[: 2 * cin].reshape(-1, 1)
    bv = b_qkv[2 * cin :].reshape(-1, 1)
    wdw = W_dw[:, 0].reshape(3 * cin, 9)
    bdwv = b_dw[2 * cin :].reshape(-1, 1)
    tempc = jnp.repeat(temperature.reshape(heads), cph).reshape(cin, 1)
    hr = (jnp.arange(cin, dtype=jnp.int32) // cph).astype(f32).reshape(cin, 1)
    hc = hr.reshape(1, cin)
    wbr = jnp.concatenate([attn1, attn2, attn3, attn4]).astype(f32)

    attn = _make_pass1(hh, ww, cin, t)(
        x0, wqk, bqk, wdw[: 2 * cin], b_dw[: 2 * cin].reshape(-1, 1), tempc)
    out = _make_pass2(hh, ww, cin, t, cph)(
        x0, wv, bv, wdw[2 * cin :], bdwv, attn, W_proj,
        b_proj.reshape(cin, 1), hr, hc, wbr)
    return out.reshape(1, cin, hh, ww)


# speedup vs baseline: 6.6322x; 1.8361x over previous
import jax
import jax.numpy as jnp
from jax import lax
from jax.experimental import pallas as pl
from jax.experimental.pallas import tpu as pltpu

_NEG = -3.0e38
_CW = 2048


def _dwconv_into(o_ref, u, wdw_ref, bdw_ref, tt, ww, hb):
    ppl = u.shape[1]
    cw = min(_CW, tt)
    qi = lax.broadcasted_iota(jnp.int32, (1, cw), 1)
    w = jnp.bitwise_and(qi, ww - 1)
    m0 = jnp.where(w != 0, 1.0, 0.0).astype(jnp.float32)
    m2 = jnp.where(w != ww - 1, 1.0, 0.0).astype(jnp.float32)
    coefs = [wdw_ref[:, j : j + 1] for j in range(9)]
    for c0 in range(0, tt, cw):
        grp = [None, None, None]
        for dx in range(3):
            for dy in range(3):
                s0 = c0 + (hb - 1 + dy) * ww + dx - 1
                if s0 < 0:
                    tap = jnp.concatenate([u[:, :1], u[:, : cw - 1]], axis=1)
                elif s0 + cw > ppl:
                    tap = jnp.concatenate([u[:, s0:], u[:, -1:]], axis=1)
                else:
                    tap = u[:, s0 : s0 + cw]
                term = coefs[dy * 3 + dx] * tap
                grp[dx] = term if grp[dx] is None else grp[dx] + term
        acc = (bdw_ref[...] + grp[1]) + (grp[0] * m0 + grp[2] * m2)
        o_ref[:, c0 : c0 + cw] = acc


def _row_valid(i, t, hh, ww, pp, hb):
    p = lax.broadcasted_iota(jnp.int32, (1, pp), 1)
    prow = i * t + lax.shift_right_logical(p, ww.bit_length() - 1)
    return jnp.where((prow >= hb) & (prow < hh + hb), 1.0, 0.0)


_HB = 8


def _halo_copies(x_hbm, xbuf, sem, j, t, nt):
    s = lax.rem(j, 2)
    hb = _HB
    cps = [(None,
            pltpu.make_async_copy(x_hbm.at[:, pl.ds(j * t, t), :],
                                  xbuf.at[s, :, pl.ds(hb, t), :], sem.at[s]))]
    if nt > 1:
        cps.append((j > 0,
                    pltpu.make_async_copy(x_hbm.at[:, pl.ds(j * t - hb, hb), :],
                                          xbuf.at[s, :, pl.ds(0, hb), :],
                                          sem.at[s])))
        cps.append((j < nt - 1,
                    pltpu.make_async_copy(x_hbm.at[:, pl.ds(j * t + t, hb), :],
                                          xbuf.at[s, :, pl.ds(t + hb, hb), :],
                                          sem.at[s])))
    return cps


def _start_window(x_hbm, xbuf, sem, j, t, nt):
    for cond, cp in _halo_copies(x_hbm, xbuf, sem, j, t, nt):
        if cond is None:
            cp.start()
        else:
            pl.when(cond)(cp.start)


def _wait_window(x_hbm, xbuf, sem, j, t, nt):
    for cond, cp in _halo_copies(x_hbm, xbuf, sem, j, t, nt):
        if cond is None:
            cp.wait()
        else:
            pl.when(cond)(cp.wait)


def _init_halo_rows(xbuf, t, nt, cin, ww):
    hb = _HB
    z = jnp.zeros((cin, hb, ww), jnp.float32)
    xbuf[0, :, 0:hb, :] = z
    if nt <= 2:
        xbuf[(nt - 1) % 2, :, t + hb : t + 2 * hb, :] = z


def _make_pass1(hh, ww, cin, t):
    nt = hh // t
    tt = t * ww
    pp = (t + 2 * _HB) * ww

    def body(x_hbm, wqk_ref, bqk_ref, wdw_ref, bdw_ref, tempc_ref, attn_ref,
             xbuf, qk_ref, qsq, ksq, sem):
        i = pl.program_id(0)

        @pl.when(i == 0)
        def _():
            _init_halo_rows(xbuf, t, nt, cin, ww)
            _start_window(x_hbm, xbuf, sem, 0, t, nt)
            attn_ref[...] = jnp.zeros_like(attn_ref)
            qsq[...] = jnp.zeros_like(qsq)
            ksq[...] = jnp.zeros_like(ksq)

        @pl.when(i + 1 < nt)
        def _():
            _start_window(x_hbm, xbuf, sem, i + 1, t, nt)

        _wait_window(x_hbm, xbuf, sem, i, t, nt)
        x = xbuf[lax.rem(i, 2)].reshape(cin, pp)
        u = jnp.dot(wqk_ref[...], x, preferred_element_type=jnp.float32)
        u = (u + bqk_ref[...]) * _row_valid(i, t, hh, ww, pp, _HB)
        _dwconv_into(qk_ref, u, wdw_ref, bdw_ref, tt, ww, _HB)
        qf = qk_ref[:cin, :]
        kf = qk_ref[cin:, :]
        dn = (((1,), (1,)), ((), ()))
        attn_ref[...] += lax.dot_general(
            qf, kf, dn, preferred_element_type=jnp.float32)
        qsq[...] += jnp.sum(qf * qf, axis=1, keepdims=True)
        ones1 = jnp.ones((1, tt), jnp.float32)
        ksq[...] += lax.dot_general(
            ones1, kf * kf, dn, preferred_element_type=jnp.float32)

        @pl.when(i == nt - 1)
        def _():
            invq = 1.0 / jnp.maximum(jnp.sqrt(qsq[...]), 1e-12)
            invk = 1.0 / jnp.maximum(jnp.sqrt(ksq[...]), 1e-12)
            attn_ref[...] = attn_ref[...] * invq * invk * tempc_ref[...]

    return pl.pallas_call(
        body,
        grid=(nt,),
        in_specs=[
            pl.BlockSpec(memory_space=pl.ANY),
            pl.BlockSpec((2 * cin, cin), lambda i: (0, 0)),
            pl.BlockSpec((2 * cin, 1), lambda i: (0, 0)),
            pl.BlockSpec((2 * cin, 9), lambda i: (0, 0)),
            pl.BlockSpec((2 * cin, 1), lambda i: (0, 0)),
            pl.BlockSpec((cin, 1), lambda i: (0, 0)),
        ],
        out_specs=pl.BlockSpec((cin, cin), lambda i: (0, 0)),
        out_shape=jax.ShapeDtypeStruct((cin, cin), jnp.float32),
        scratch_shapes=[
            pltpu.VMEM((2, cin, t + 2 * _HB, ww), jnp.float32),
            pltpu.VMEM((2 * cin, tt), jnp.float32),
            pltpu.VMEM((cin, 1), jnp.float32),
            pltpu.VMEM((1, cin), jnp.float32),
            pltpu.SemaphoreType.DMA((2,)),
        ],
        compiler_params=pltpu.CompilerParams(
            dimension_semantics=("arbitrary",)),
    )


def _make_pass2(hh, ww, cin, t, cph):
    nt = hh // t
    tt = t * ww
    pp = (t + 2 * _HB) * ww

    def body(x_hbm, wv_ref, bv_ref, wdw_ref, bdw_ref, attn_in, wproj_ref,
             bproj_ref, hr_ref, hc_ref, wbr_ref, o_ref, xbuf, v_ref, m_ref,
             sem):
        i = pl.program_id(0)

        @pl.when(i == 0)
        def _():
            _init_halo_rows(xbuf, t, nt, cin, ww)
            _start_window(x_hbm, xbuf, sem, 0, t, nt)

        @pl.when(i + 1 < nt)
        def _():
            _start_window(x_hbm, xbuf, sem, i + 1, t, nt)

        _wait_window(x_hbm, xbuf, sem, i, t, nt)

        @pl.when(i == 0)
        def _():
            bd = hr_ref[...] == hc_ref[...]
            a = jnp.where(bd, attn_in[...], _NEG)
            x1 = a[:, :, None]
            x2 = a[:, None, :]
            n = a.shape[0]
            lt = (lax.broadcasted_iota(jnp.int32, (1, n, n), 1)
                  < lax.broadcasted_iota(jnp.int32, (1, n, n), 2))
            cnt = (x1 > x2) | ((x1 == x2) & lt)
            rank = jnp.sum(cnt.astype(jnp.float32), axis=1)

            def smax(kk):
                sel = bd & (rank < kk)
                m = jnp.max(jnp.where(sel, a, _NEG), axis=1, keepdims=True)
                e = jnp.where(sel, jnp.exp(a - m), 0.0)
                return e / jnp.sum(e, axis=1, keepdims=True)

            bdm = (wbr_ref[0] * smax(float(cph // 2))
                   + wbr_ref[1] * smax(float(cph * 2 // 3))
                   + (wbr_ref[2] + wbr_ref[3]) * smax(float(cph * 3 // 4)))
            m_ref[...] = jnp.dot(wproj_ref[...], bdm,
                                 preferred_element_type=jnp.float32)

        x = xbuf[lax.rem(i, 2)].reshape(cin, pp)
        u = jnp.dot(wv_ref[...], x, preferred_element_type=jnp.float32)
        u = (u + bv_ref[...]) * _row_valid(i, t, hh, ww, pp, _HB)
        _dwconv_into(v_ref, u, wdw_ref, bdw_ref, tt, ww, _HB)
        o = jnp.dot(m_ref[...], v_ref[...],
                    preferred_element_type=jnp.float32) + bproj_ref[...]
        o_ref[...] = o.reshape(cin, t, ww)

    return pl.pallas_call(
        body,
        grid=(nt,),
        in_specs=[
            pl.BlockSpec(memory_space=pl.ANY),
            pl.BlockSpec((cin, cin), lambda i: (0, 0)),
            pl.BlockSpec((cin, 1), lambda i: (0, 0)),
            pl.BlockSpec((cin, 9), lambda i: (0, 0)),
            pl.BlockSpec((cin, 1), lambda i: (0, 0)),
            pl.BlockSpec((cin, cin), lambda i: (0, 0)),
            pl.BlockSpec((cin, cin), lambda i: (0, 0)),
            pl.BlockSpec((cin, 1), lambda i: (0, 0)),
            pl.BlockSpec((cin, 1), lambda i: (0, 0)),
            pl.BlockSpec((1, cin), lambda i: (0, 0)),
            pl.BlockSpec(memory_space=pltpu.SMEM),
        ],
        out_specs=pl.BlockSpec((cin, t, ww), lambda i: (0, i, 0)),
        out_shape=jax.ShapeDtypeStruct((cin, hh, ww), jnp.float32),
        scratch_shapes=[
            pltpu.VMEM((2, cin, t + 2 * _HB, ww), jnp.float32),
            pltpu.VMEM((cin, tt), jnp.float32),
            pltpu.VMEM((cin, cin), jnp.float32),
            pltpu.SemaphoreType.DMA((2,)),
        ],
        compiler_params=pltpu.CompilerParams(
            dimension_semantics=("arbitrary",)),
    )


def kernel(x, temperature, W_qkv, b_qkv, W_dw, b_dw, W_proj, b_proj,
           attn1, attn2, attn3, attn4):
    _, cin, hh, ww = x.shape
    heads = temperature.shape[0]
    cph = cin // heads
    t = 32 if hh % 32 == 0 else hh
    f32 = jnp.float32

    x0 = x[0]
    wqk = W_qkv[: 2 * cin]
    wv = W_qkv[2 * cin :]
    bqk = b_qkv[: 2 * cin].reshape(-1, 1)
    bv = b_qkv[2 * cin :].reshape(-1, 1)
    wdw = W_dw[:, 0].reshape(3 * cin, 9)
    bdwv = b_dw[2 * cin :].reshape(-1, 1)
    tempc = jnp.repeat(temperature.reshape(heads), cph).reshape(cin, 1)
    hr = (jnp.arange(cin, dtype=jnp.int32) // cph).astype(f32).reshape(cin, 1)
    hc = hr.reshape(1, cin)
    wbr = jnp.concatenate([attn1, attn2, attn3, attn4]).astype(f32)

    attn = _make_pass1(hh, ww, cin, t)(
        x0, wqk, bqk, wdw[: 2 * cin], b_dw[: 2 * cin].reshape(-1, 1), tempc)
    out = _make_pass2(hh, ww, cin, t, cph)(
        x0, wv, bv, wdw[2 * cin :], bdwv, attn, W_proj,
        b_proj.reshape(cin, 1), hr, hc, wbr)
    return out.reshape(1, cin, hh, ww)
